# baseline (speedup 1.0000x reference)
import functools

import jax
import jax.numpy as jnp
from jax import lax
from jax.experimental import pallas as pl
from jax.experimental.pallas import tpu as pltpu
from jax.experimental.pallas import tpu_sc as plsc

_C = 16384
_U = 8


def _sc_broadcast_add(inputs, table):
    batch, seq, dim = inputs.shape
    n = seq * dim
    info = plsc.get_sparse_core_info()
    nc, ns, nl = info.num_cores, info.num_subcores, info.num_lanes
    nw = nc * ns
    n_per_w = n // nw
    n_chunks = n_per_w // _C
    n_iter = n_chunks * batch
    mesh = plsc.VectorSubcoreMesh(core_axis_name="c", subcore_axis_name="s")

    @functools.partial(
        pl.kernel,
        mesh=mesh,
        out_type=jax.ShapeDtypeStruct((batch, n), jnp.float32),
        scratch_types=(
            [pltpu.VMEM((_C,), jnp.float32) for _ in range(4)]
            + [pltpu.VMEM((_C,), jnp.float32) for _ in range(2)]
            + [pltpu.SemaphoreType.DMA for _ in range(10)]
        ),
    )
    def k(in_hbm, tab_hbm, out_hbm, *scr):
        bufs = scr[0:4]
        tabs = scr[4:6]
        lsem = scr[6:10]
        ssem = scr[10:14]
        tsem = scr[14:16]
        wid = lax.axis_index("s") * nc + lax.axis_index("c")
        w0 = wid * n_per_w

        def in_slice(i):
            c, b = divmod(i, batch)
            return in_hbm.at[b, pl.ds(w0 + c * _C, _C)]

        def out_slice(i):
            c, b = divmod(i, batch)
            return out_hbm.at[b, pl.ds(w0 + c * _C, _C)]

        def tab_slice(c):
            return tab_hbm.at[pl.ds(w0 + c * _C, _C)]

        def load(i):
            return pltpu.make_async_copy(in_slice(i), bufs[i % 4], lsem[i % 4])

        def store(i):
            return pltpu.make_async_copy(bufs[i % 4], out_slice(i), ssem[i % 4])

        def tabcp(c):
            return pltpu.make_async_copy(tab_slice(c), tabs[c % 2], tsem[c % 2])

        tabcp(0).start()
        if n_chunks > 1:
            tabcp(1).start()
        for i in range(min(3, n_iter)):
            load(i).start()
        for i in range(n_iter):
            c, b = divmod(i, batch)
            load(i).wait()
            if b == 0:
                tabcp(c).wait()
            buf = bufs[i % 4]
            tv = tabs[c % 2]

            def body(i2, carry, buf=buf, tv=tv):
                for u in range(_U):
                    sl = pl.ds((i2 * _U + u) * nl, nl)
                    buf[sl] = buf[sl] + tv[sl]
                return carry

            lax.fori_loop(0, _C // (nl * _U), body, 0)
            store(i).start()
            if b == batch - 1 and c + 2 < n_chunks:
                tabcp(c + 2).start()
            if i + 3 < n_iter:
                if i >= 1:
                    store(i - 1).wait()
                load(i + 3).start()
        for i in range(max(0, n_iter - 4), n_iter):
            store(i).wait()

    out = k(inputs.reshape(batch, n), table.reshape(n))
    return out.reshape(batch, seq, dim)


def kernel(inputs, position_table):
    return _sc_broadcast_add(inputs, position_table)


# speedup vs baseline: 2.3575x; 2.3575x over previous
import functools

import jax
import jax.numpy as jnp
from jax import lax
from jax.experimental import pallas as pl
from jax.experimental.pallas import tpu as pltpu
from jax.experimental.pallas import tpu_sc as plsc

_P = 16


def _sc_broadcast_add(inputs, table):
    batch, seq, dim = inputs.shape
    info = plsc.get_sparse_core_info()
    nc, ns, nl = info.num_cores, info.num_subcores, info.num_lanes
    nw = nc * ns
    s_per_w = seq // nw
    n_chunks = s_per_w // _P
    n_iter = n_chunks * batch
    mesh = plsc.VectorSubcoreMesh(core_axis_name="c", subcore_axis_name="s")

    @functools.partial(
        pl.kernel,
        mesh=mesh,
        out_type=jax.ShapeDtypeStruct((batch, seq, dim), jnp.float32),
        scratch_types=(
            [pltpu.VMEM((_P, dim), jnp.float32) for _ in range(4)]
            + [pltpu.VMEM((_P, dim), jnp.float32) for _ in range(2)]
            + [pltpu.SemaphoreType.DMA for _ in range(10)]
        ),
    )
    def k(in_hbm, tab_hbm, out_hbm, *scr):
        bufs = scr[0:4]
        tabs = scr[4:6]
        lsem = scr[6:10]
        ssem = scr[10:14]
        tsem = scr[14:16]
        wid = lax.axis_index("s") * nc + lax.axis_index("c")
        s0 = wid * s_per_w

        def in_slice(i):
            c, b = divmod(i, batch)
            return in_hbm.at[b, pl.ds(s0 + c * _P, _P)]

        def out_slice(i):
            c, b = divmod(i, batch)
            return out_hbm.at[b, pl.ds(s0 + c * _P, _P)]

        def tab_slice(c):
            return tab_hbm.at[pl.ds(s0 + c * _P, _P)]

        def load(i):
            return pltpu.make_async_copy(in_slice(i), bufs[i % 4], lsem[i % 4])

        def store(i):
            return pltpu.make_async_copy(bufs[i % 4], out_slice(i), ssem[i % 4])

        def tabcp(c):
            return pltpu.make_async_copy(tab_slice(c), tabs[c % 2], tsem[c % 2])

        tabcp(0).start()
        if n_chunks > 1:
            tabcp(1).start()
        for i in range(min(3, n_iter)):
            load(i).start()
        for i in range(n_iter):
            c, b = divmod(i, batch)
            load(i).wait()
            if b == 0:
                tabcp(c).wait()
            buf = bufs[i % 4]
            tv = tabs[c % 2]

            def body(i2, carry, buf=buf, tv=tv):
                r = i2 // 2
                col = (i2 % 2) * (dim // 2)
                for u in range(dim // (2 * nl)):
                    sl = pl.ds(col + u * nl, nl)
                    buf[r, sl] = buf[r, sl] + tv[r, sl]
                return carry

            lax.fori_loop(0, 2 * _P, body, 0)
            store(i).start()
            if b == batch - 1 and c + 2 < n_chunks:
                tabcp(c + 2).start()
            if i + 3 < n_iter:
                if i >= 1:
                    store(i - 1).wait()
                load(i + 3).start()
        for i in range(max(0, n_iter - 4), n_iter):
            store(i).wait()

    return k(inputs, table)


def kernel(inputs, position_table):
    return _sc_broadcast_add(inputs, position_table)
